# baseline (speedup 1.0000x reference)
import jax
import jax.numpy as jnp
from jax.experimental import pallas as pl
from jax.experimental.pallas import tpu as pltpu


def kernel(tokens, spatial_shape, fc1_w, fc1_b, fc2_w, fc2_b):
    B, N, C = tokens.shape
    flat = tokens.reshape(B * N, C)
    R = B * N
    K = 2
    CHUNK = R // K

    def body(sv_ref, x_hbm, o_hbm, sem_in, sem_out, buf):
        valid = sv_ref[0] * sv_ref[1] == N

        @pl.when(valid)
        def _copy():
            for i in range(K):
                pltpu.make_async_copy(
                    x_hbm.at[pl.ds(i * CHUNK, CHUNK)],
                    buf.at[pl.ds(i * CHUNK, CHUNK)],
                    sem_in.at[i],
                ).start()
            for i in range(K):
                pltpu.make_async_copy(
                    x_hbm.at[pl.ds(i * CHUNK, CHUNK)],
                    buf.at[pl.ds(i * CHUNK, CHUNK)],
                    sem_in.at[i],
                ).wait()
                pltpu.make_async_copy(
                    buf.at[pl.ds(i * CHUNK, CHUNK)],
                    o_hbm.at[pl.ds(i * CHUNK, CHUNK)],
                    sem_out,
                ).start()
            for i in range(K):
                pltpu.make_async_copy(
                    buf.at[pl.ds(i * CHUNK, CHUNK)],
                    o_hbm.at[pl.ds(i * CHUNK, CHUNK)],
                    sem_out,
                ).wait()

        @pl.when(jnp.logical_not(valid))
        def _nan_fill():
            buf[pl.ds(0, CHUNK)] = jnp.full((CHUNK, C), jnp.nan, jnp.float32)
            for i in range(K):
                cp = pltpu.make_async_copy(
                    buf.at[pl.ds(0, CHUNK)],
                    o_hbm.at[pl.ds(i * CHUNK, CHUNK)],
                    sem_out,
                )
                cp.start()
                cp.wait()

    out = pl.pallas_call(
        body,
        in_specs=[
            pl.BlockSpec(memory_space=pltpu.MemorySpace.SMEM),
            pl.BlockSpec(memory_space=pltpu.MemorySpace.HBM),
        ],
        out_specs=pl.BlockSpec(memory_space=pltpu.MemorySpace.HBM),
        out_shape=jax.ShapeDtypeStruct((R, C), jnp.float32),
        scratch_shapes=[
            pltpu.SemaphoreType.DMA((K,)),
            pltpu.SemaphoreType.DMA,
            pltpu.VMEM((R, C), jnp.float32),
        ],
    )(spatial_shape, flat)
    return out.reshape(B, N, C)


# speedup vs baseline: 1.0699x; 1.0699x over previous
import jax
import jax.numpy as jnp
from jax.experimental import pallas as pl
from jax.experimental.pallas import tpu as pltpu


def kernel(tokens, spatial_shape, fc1_w, fc1_b, fc2_w, fc2_b):
    B, N, C = tokens.shape
    flat = tokens.reshape(B * N, C)
    R = B * N
    BLK = 2048

    def body(sv_ref, x_ref, o_ref):
        valid = sv_ref[0] * sv_ref[1] == N
        o_ref[...] = jnp.where(valid, x_ref[...], jnp.float32(jnp.nan))

    out = pl.pallas_call(
        body,
        grid=(R // BLK,),
        in_specs=[
            pl.BlockSpec(memory_space=pltpu.MemorySpace.SMEM),
            pl.BlockSpec((BLK, C), lambda i: (i, 0)),
        ],
        out_specs=pl.BlockSpec((BLK, C), lambda i: (i, 0)),
        out_shape=jax.ShapeDtypeStruct((R, C), jnp.float32),
    )(spatial_shape, flat)
    return out.reshape(B, N, C)
